# baseline (speedup 1.0000x reference)
import functools

import jax
import jax.numpy as jnp
from jax import lax
from jax.experimental import pallas as pl
from jax.experimental.pallas import tpu as pltpu
from jax.experimental.pallas import tpu_sc as plsc

CLASS_NUM = 1000
FEATURES_NUM = 512
BATCH = 4096

_NC = 2
_NS = 16
_NW = _NC * _NS
_L = 16
_ROWS_PER_W = BATCH // _NW
_GROUP = 16
_NGROUPS = _ROWS_PER_W // _GROUP
_KV = FEATURES_NUM // _L
_KVB = FEATURES_NUM // (2 * _L)
_W32 = FEATURES_NUM // 2
_CPAD = 1024


def _rsqrt16(x):
    i = plsc.bitcast(x, jnp.int32)
    i = jnp.int32(0x5F3759DF) - (i >> 1)
    y = plsc.bitcast(i, jnp.float32)
    for _ in range(3):
        y = y * (1.5 - 0.5 * x * y * y)
    return y


def _sc_partials(features, labels, center):
    mesh = plsc.VectorSubcoreMesh(core_axis_name="c", subcore_axis_name="s")

    @functools.partial(
        pl.kernel,
        mesh=mesh,
        compiler_params=pltpu.CompilerParams(
            needs_layout_passes=False,
            disable_bounds_checks=True,
            disable_semaphore_checks=True,
            skip_device_barrier=True,
        ),
        out_type=(
            jax.ShapeDtypeStruct((_NC, _CPAD), jnp.float32),
            jax.ShapeDtypeStruct((_NC, _CPAD), jnp.float32),
        ),
        scratch_types=[
            pltpu.VMEM((_ROWS_PER_W,), jnp.int32),
            pltpu.VMEM((2, _GROUP, _W32), jnp.int32),
            pltpu.VMEM((2, _GROUP, _W32), jnp.int32),
            pltpu.VMEM((_ROWS_PER_W,), jnp.float32),
            pltpu.VMEM((_ROWS_PER_W,), jnp.float32),
            pltpu.VMEM((_CPAD,), jnp.float32),
            pltpu.VMEM((_GROUP, 17), jnp.float32),
            pltpu.SemaphoreType.DMA,
            pltpu.SemaphoreType.DMA,
            pltpu.SemaphoreType.DMA,
            pltpu.SemaphoreType.DMA,
            pltpu.VMEM_SHARED((_CPAD,), jnp.float32),
            pltpu.VMEM_SHARED((_CPAD,), jnp.float32),
        ],
    )
    def k(feat_hbm, lab_hbm, cen_hbm, s_out, cnt_out,
          labels_v, feat_v, rows_v, dist_v, ones_v, zbuf, d2t,
          semf0, semf1, semr0, semr1, sh_s, sh_cnt):
        cid = lax.axis_index("c")
        sid = lax.axis_index("s")
        wid = sid * _NC + cid
        base = wid * _ROWS_PER_W
        semf = (semf0, semf1)
        semr = (semr0, semr1)

        pltpu.sync_copy(lab_hbm.at[pl.ds(base, _ROWS_PER_W)], labels_v)

        def start(g, b):
            rbase = base + g * _GROUP
            pltpu.async_copy(
                feat_hbm.at[pl.ds(rbase, _GROUP)], feat_v.at[b], semf[b])
            pltpu.async_copy(
                cen_hbm.at[labels_v.at[pl.ds(g * _GROUP, _GROUP)]],
                rows_v.at[b], semr[b])

        def wait(b):
            pltpu.make_async_copy(
                feat_hbm.at[pl.ds(0, _GROUP)], feat_v.at[b], semf[b]).wait()
            pltpu.make_async_copy(
                feat_hbm.at[pl.ds(0, _GROUP)], rows_v.at[b], semr[b]).wait()

        def compute(g, b):
            @plsc.parallel_loop(0, _GROUP, 1, unroll=2)
            def _(j):
                accs = [jnp.zeros((_L,), jnp.float32) for _ in range(4)]
                for kk in range(_KVB):
                    f = plsc.bitcast(
                        feat_v[b, j, pl.ds(kk * _L, _L)], jnp.bfloat16)
                    c = plsc.bitcast(
                        rows_v[b, j, pl.ds(kk * _L, _L)], jnp.bfloat16)
                    d = f - c
                    lo, hi = plsc.unpack(d, format=plsc.PackFormat.INTERLEAVED)
                    a = (2 * kk) % 4
                    accs[a] = accs[a] + lo * lo
                    accs[a + 1] = accs[a + 1] + hi * hi
                d2t[j, pl.ds(0, _L)] = (accs[0] + accs[1]) + (accs[2] + accs[3])
            ri = lax.iota(jnp.int32, _L)
            tot = jnp.zeros((_L,), jnp.float32)
            for kk in range(_L):
                tot = tot + plsc.load_gather(
                    d2t, [ri, jnp.full((_L,), kk, jnp.int32)])
            dist = tot * _rsqrt16(jnp.maximum(tot, 1e-30))
            dist_v[pl.ds(g * _GROUP, _GROUP)] = dist

        start(0, 0)
        start(1, 1)

        zero16 = jnp.zeros((_L,), jnp.float32)
        for i in range(_CPAD // _L):
            zbuf[pl.ds(i * _L, _L)] = zero16
        one16 = jnp.full((_L,), 1.0, jnp.float32)
        for i in range(_ROWS_PER_W // _L):
            ones_v[pl.ds(i * _L, _L)] = one16

        @pl.when(sid == 0)
        def _():
            pltpu.sync_copy(zbuf, sh_s)
            pltpu.sync_copy(zbuf, sh_cnt)

        plsc.subcore_barrier()

        n_it = _NGROUPS // 2

        def it_body(it, carry):
            g0 = it * 2
            wait(0)
            compute(g0, 0)

            @pl.when(it < n_it - 1)
            def _():
                start(g0 + 2, 0)

            wait(1)
            compute(g0 + 1, 1)

            @pl.when(it < n_it - 1)
            def _():
                start(g0 + 3, 1)

            return carry

        lax.fori_loop(0, n_it, it_body, 0)

        pltpu.sync_copy(dist_v, sh_s.at[labels_v], add=True)
        pltpu.sync_copy(ones_v, sh_cnt.at[labels_v], add=True)

        plsc.subcore_barrier()

        @pl.when(sid == 0)
        def _():
            pltpu.sync_copy(sh_s, s_out.at[cid])
            pltpu.sync_copy(sh_cnt, cnt_out.at[cid])

    return k(features, labels, center)


def _combine_kernel(s_ref, c_ref, o_ref):
    s = s_ref[0:1, :] + s_ref[1:2, :]
    c = c_ref[0:1, :] + c_ref[1:2, :]
    o_ref[...] = jnp.sum(s / jnp.maximum(c, 1.0)).reshape(1, 1)


def _combine(s_part, cnt_part):
    out = pl.pallas_call(
        _combine_kernel,
        out_shape=jax.ShapeDtypeStruct((1, 1), jnp.float32),
    )(s_part, cnt_part)
    return out[0, 0]


def kernel(features, labels, center):
    labels = labels.astype(jnp.int32)
    fpk = lax.bitcast_convert_type(
        features.astype(jnp.bfloat16).reshape(BATCH, _W32, 2), jnp.int32)
    cpk = lax.bitcast_convert_type(
        center.astype(jnp.bfloat16).reshape(CLASS_NUM, _W32, 2), jnp.int32)
    s_part, cnt_part = _sc_partials(fpk, labels, cpk)
    return _combine(s_part, cnt_part)


# speedup vs baseline: 2.2387x; 2.2387x over previous
import functools

import jax
import jax.numpy as jnp
from jax import lax
from jax.experimental import pallas as pl
from jax.experimental.pallas import tpu as pltpu
from jax.experimental.pallas import tpu_sc as plsc

CLASS_NUM = 1000
FEATURES_NUM = 512
BATCH = 4096

_NC = 2
_NS = 16
_NW = _NC * _NS
_L = 16
_ROWS_PER_W = BATCH // _NW
_GROUP = 16
_NGROUPS = _ROWS_PER_W // _GROUP
_KV = FEATURES_NUM // _L
_KVB = FEATURES_NUM // (2 * _L)
_W32 = FEATURES_NUM // 2
_CPAD = 1024


def _rsqrt16(x):
    i = plsc.bitcast(x, jnp.int32)
    i = jnp.int32(0x5F3759DF) - (i >> 1)
    y = plsc.bitcast(i, jnp.float32)
    for _ in range(3):
        y = y * (1.5 - 0.5 * x * y * y)
    return y


def _sc_partials(features, labels, center):
    mesh = plsc.VectorSubcoreMesh(core_axis_name="c", subcore_axis_name="s")

    @functools.partial(
        pl.kernel,
        mesh=mesh,
        compiler_params=pltpu.CompilerParams(
            needs_layout_passes=False,
            disable_bounds_checks=True,
            disable_semaphore_checks=True,
            skip_device_barrier=True,
        ),
        out_type=(
            jax.ShapeDtypeStruct((_NC, _CPAD), jnp.float32),
            jax.ShapeDtypeStruct((_NC, _CPAD), jnp.float32),
        ),
        scratch_types=[
            pltpu.VMEM((_ROWS_PER_W,), jnp.int32),
            pltpu.VMEM((2, _GROUP, FEATURES_NUM), jnp.float32),
            pltpu.VMEM((2, _GROUP, FEATURES_NUM), jnp.float32),
            pltpu.VMEM((_ROWS_PER_W,), jnp.float32),
            pltpu.VMEM((_ROWS_PER_W,), jnp.float32),
            pltpu.VMEM((_CPAD,), jnp.float32),
            pltpu.VMEM((_GROUP, 17), jnp.float32),
            pltpu.SemaphoreType.DMA,
            pltpu.SemaphoreType.DMA,
            pltpu.SemaphoreType.DMA,
            pltpu.SemaphoreType.DMA,
            pltpu.VMEM_SHARED((_CPAD,), jnp.float32),
            pltpu.VMEM_SHARED((_CPAD,), jnp.float32),
        ],
    )
    def k(feat_hbm, lab_hbm, cen_hbm, s_out, cnt_out,
          labels_v, feat_v, rows_v, dist_v, ones_v, zbuf, d2t,
          semf0, semf1, semr0, semr1, sh_s, sh_cnt):
        cid = lax.axis_index("c")
        sid = lax.axis_index("s")
        wid = sid * _NC + cid
        base = wid * _ROWS_PER_W
        semf = (semf0, semf1)
        semr = (semr0, semr1)

        pltpu.sync_copy(lab_hbm.at[pl.ds(base, _ROWS_PER_W)], labels_v)

        def start(g, b):
            rbase = base + g * _GROUP
            pltpu.async_copy(
                feat_hbm.at[pl.ds(rbase, _GROUP)], feat_v.at[b], semf[b])
            pltpu.async_copy(
                cen_hbm.at[labels_v.at[pl.ds(g * _GROUP, _GROUP)]],
                rows_v.at[b], semr[b])

        def wait(b):
            pltpu.make_async_copy(
                feat_hbm.at[pl.ds(0, _GROUP)], feat_v.at[b], semf[b]).wait()
            pltpu.make_async_copy(
                feat_hbm.at[pl.ds(0, _GROUP)], rows_v.at[b], semr[b]).wait()

        def compute(g, b):
            @plsc.parallel_loop(0, _GROUP, 1, unroll=2)
            def _(j):
                accs = [jnp.zeros((_L,), jnp.float32) for _ in range(4)]
                for kk in range(_KV):
                    f = feat_v[b, j, pl.ds(kk * _L, _L)]
                    c = rows_v[b, j, pl.ds(kk * _L, _L)]
                    d = f - c
                    accs[kk % 4] = accs[kk % 4] + d * d
                d2t[j, pl.ds(0, _L)] = (accs[0] + accs[1]) + (accs[2] + accs[3])
            ri = lax.iota(jnp.int32, _L)
            tot = jnp.zeros((_L,), jnp.float32)
            for kk in range(_L):
                tot = tot + plsc.load_gather(
                    d2t, [ri, jnp.full((_L,), kk, jnp.int32)])
            dist = tot * _rsqrt16(jnp.maximum(tot, 1e-30))
            dist_v[pl.ds(g * _GROUP, _GROUP)] = dist

        start(0, 0)
        start(1, 1)

        zero16 = jnp.zeros((_L,), jnp.float32)
        for i in range(_CPAD // _L):
            zbuf[pl.ds(i * _L, _L)] = zero16
        one16 = jnp.full((_L,), 1.0, jnp.float32)
        for i in range(_ROWS_PER_W // _L):
            ones_v[pl.ds(i * _L, _L)] = one16

        @pl.when(sid == 0)
        def _():
            pltpu.sync_copy(zbuf, sh_s)
            pltpu.sync_copy(zbuf, sh_cnt)

        plsc.subcore_barrier()

        n_it = _NGROUPS // 2

        def it_body(it, carry):
            g0 = it * 2
            wait(0)
            compute(g0, 0)

            @pl.when(it < n_it - 1)
            def _():
                start(g0 + 2, 0)

            wait(1)
            compute(g0 + 1, 1)

            @pl.when(it < n_it - 1)
            def _():
                start(g0 + 3, 1)

            return carry

        lax.fori_loop(0, n_it, it_body, 0)

        pltpu.sync_copy(dist_v, sh_s.at[labels_v], add=True)
        pltpu.sync_copy(ones_v, sh_cnt.at[labels_v], add=True)

        plsc.subcore_barrier()

        @pl.when(sid == 0)
        def _():
            pltpu.sync_copy(sh_s, s_out.at[cid])
            pltpu.sync_copy(sh_cnt, cnt_out.at[cid])

    return k(features, labels, center)


def _combine_kernel(s_ref, c_ref, o_ref):
    s = s_ref[0:1, :] + s_ref[1:2, :]
    c = c_ref[0:1, :] + c_ref[1:2, :]
    o_ref[...] = jnp.sum(s / jnp.maximum(c, 1.0)).reshape(1, 1)


def _combine(s_part, cnt_part):
    out = pl.pallas_call(
        _combine_kernel,
        out_shape=jax.ShapeDtypeStruct((1, 1), jnp.float32),
    )(s_part, cnt_part)
    return out[0, 0]


def kernel(features, labels, center):
    labels = labels.astype(jnp.int32)
    s_part, cnt_part = _sc_partials(features, labels, center)
    return _combine(s_part, cnt_part)
